# baseline (speedup 1.0000x reference)
import jax
import jax.numpy as jnp
from jax import lax
from jax.experimental import pallas as pl
from jax.experimental.pallas import tpu as pltpu
from jax.experimental.pallas import tpu_sc as plsc

D = 4096
NC = 2
NS = 16
NW = NC * NS

CHUNK = 8


def _gather_kernel(B, b_per_w):
    n_chunks = b_per_w // CHUNK
    mesh = plsc.VectorSubcoreMesh(
        core_axis_name="c", subcore_axis_name="s", num_cores=NC, num_subcores=NS
    )

    def body(pos_hbm, pe_hbm, out_hbm, idx_v, rows0, rows1, sem0, sem1):
        wid = lax.axis_index("s") * NC + lax.axis_index("c")
        base = wid * b_per_w
        pltpu.sync_copy(pos_hbm.at[pl.ds(base, b_per_w)], idx_v)

        bufs = (rows0, rows1)
        sems = (sem0, sem1)

        def start_gather(g, slot):
            pltpu.async_copy(
                pe_hbm.at[idx_v.at[pl.ds(g * CHUNK, CHUNK)]], bufs[slot], sems[slot]
            )

        def wait_gather(g, slot):
            pltpu.make_async_copy(
                pe_hbm.at[idx_v.at[pl.ds(g * CHUNK, CHUNK)]], bufs[slot], sems[slot]
            ).wait()

        start_gather(0, 0)

        def step(g, _):
            def run(slot):
                wait_gather(g, slot)

                @pl.when(g + 1 < n_chunks)
                def _():
                    start_gather(g + 1, 1 - slot)

                pltpu.sync_copy(bufs[slot], out_hbm.at[pl.ds(base + g * CHUNK, CHUNK)])

            lax.cond(g % 2 == 0, lambda: run(0), lambda: run(1))
            return _

        lax.fori_loop(0, n_chunks, step, 0)

    return pl.kernel(
        body,
        out_type=jax.ShapeDtypeStruct((B, D), jnp.float32),
        mesh=mesh,
        scratch_types=[
            pltpu.VMEM((b_per_w,), jnp.int32),
            pltpu.VMEM((CHUNK, D), jnp.float32),
            pltpu.VMEM((CHUNK, D), jnp.float32),
            pltpu.SemaphoreType.DMA,
            pltpu.SemaphoreType.DMA,
        ],
    )


def kernel(pos, pe):
    batch, seq = pos.shape
    B = batch * seq
    flat_pos = pos.reshape(B).astype(jnp.int32)
    out = _gather_kernel(B, B // NW)(flat_pos, pe)
    return out.reshape(batch, seq, D)


# speedup vs baseline: 1.0127x; 1.0127x over previous
import jax
import jax.numpy as jnp
from jax import lax
from jax.experimental import pallas as pl
from jax.experimental.pallas import tpu as pltpu
from jax.experimental.pallas import tpu_sc as plsc

D = 4096
NC = 2
NS = 16
NW = NC * NS

CHUNK = 8


def _gather_kernel(B, b_per_w):
    n_chunks = b_per_w // CHUNK
    mesh = plsc.VectorSubcoreMesh(
        core_axis_name="c", subcore_axis_name="s", num_cores=NC, num_subcores=NS
    )

    def body(pos_hbm, pe_hbm, out_hbm, idx_v, rows0, rows1, rows2, sem0, sem1, sem2):
        wid = lax.axis_index("s") * NC + lax.axis_index("c")
        base = wid * b_per_w
        pltpu.sync_copy(pos_hbm.at[pl.ds(base, b_per_w)], idx_v)

        bufs = (rows0, rows1, rows2)
        sems = (sem0, sem1, sem2)

        def start_gather(g, slot):
            pltpu.async_copy(
                pe_hbm.at[idx_v.at[pl.ds(g * CHUNK, CHUNK)]], bufs[slot], sems[slot]
            )

        def wait_gather(g, slot):
            pltpu.make_async_copy(
                pe_hbm.at[idx_v.at[pl.ds(g * CHUNK, CHUNK)]], bufs[slot], sems[slot]
            ).wait()

        start_gather(0, 0)
        start_gather(1, 1)

        def step(g, _):
            def run(slot):
                wait_gather(g, slot)

                @pl.when(g + 2 < n_chunks)
                def _():
                    start_gather(g + 2, (slot + 2) % 3)

                pltpu.sync_copy(bufs[slot], out_hbm.at[pl.ds(base + g * CHUNK, CHUNK)])

            lax.switch(g % 3, [lambda: run(0), lambda: run(1), lambda: run(2)])
            return _

        lax.fori_loop(0, n_chunks, step, 0)

    return pl.kernel(
        body,
        out_type=jax.ShapeDtypeStruct((B, D), jnp.float32),
        mesh=mesh,
        scratch_types=[
            pltpu.VMEM((b_per_w,), jnp.int32),
            pltpu.VMEM((CHUNK, D), jnp.float32),
            pltpu.VMEM((CHUNK, D), jnp.float32),
            pltpu.VMEM((CHUNK, D), jnp.float32),
            pltpu.SemaphoreType.DMA,
            pltpu.SemaphoreType.DMA,
            pltpu.SemaphoreType.DMA,
        ],
    )


def kernel(pos, pe):
    batch, seq = pos.shape
    B = batch * seq
    flat_pos = pos.reshape(B).astype(jnp.int32)
    out = _gather_kernel(B, B // NW)(flat_pos, pe)
    return out.reshape(batch, seq, D)
